# baseline (speedup 1.0000x reference)
import jax
import jax.numpy as jnp
from jax.experimental import pallas as pl
from jax.experimental.pallas import tpu as pltpu

_B, _F, _H, _W, _C = 1, 64, 32, 32, 768
_T = _H * _W
_EPS = 1e-12

_FB = 8
_HB = 16
_NH = _H // _HB
_NFS = _F // _FB


def _body(g_ref, row_ref, col_ref, tte_ref, w_ref, b_ref, out_ref,
          acc0_ref, acc1_ref):
    fs = pl.program_id(0)
    hb = pl.program_id(1)
    part = g_ref[0]
    for i in range(1, _FB):
        part = part + g_ref[i]

    def accum(acc_ref):
        @pl.when(fs == 0)
        def _init():
            acc_ref[...] = part

        @pl.when(fs > 0)
        def _accum():
            acc_ref[...] += part

        @pl.when(fs == _NFS - 1)
        def _finish():
            x = acc_ref[...] * (1.0 / _F)
            row = row_ref[pl.ds(hb * _HB, _HB)]
            x = x + row[:, None, :] + col_ref[...][None, :, :]
            x = x + tte_ref[...][None, :, :]
            mu = jnp.mean(x, axis=-1, keepdims=True)
            var = jnp.mean(jnp.square(x - mu), axis=-1, keepdims=True)
            y = (x - mu) * jax.lax.rsqrt(var + _EPS)
            out_ref[...] = y * w_ref[...][None, :, :] + b_ref[...][None, :, :]

    @pl.when(hb == 0)
    def _h0():
        accum(acc0_ref)

    @pl.when(hb == 1)
    def _h1():
        accum(acc1_ref)


def kernel(grid, row_emb, col_emb, token_type_emb, ln_weight, ln_bias):
    g = grid.reshape(_F, _H, _W, _C)
    w2 = ln_weight.reshape(1, _C)
    b2 = ln_bias.reshape(1, _C)
    out = pl.pallas_call(
        _body,
        grid=(_NFS, _NH),
        in_specs=[
            pl.BlockSpec((_FB, _HB, _W, _C), lambda fs, hb: (fs, hb, 0, 0)),
            pl.BlockSpec((_H, _C), lambda fs, hb: (0, 0)),
            pl.BlockSpec((_W, _C), lambda fs, hb: (0, 0)),
            pl.BlockSpec((1, _C), lambda fs, hb: (0, 0)),
            pl.BlockSpec((1, _C), lambda fs, hb: (0, 0)),
            pl.BlockSpec((1, _C), lambda fs, hb: (0, 0)),
        ],
        out_specs=pl.BlockSpec((_HB, _W, _C), lambda fs, hb: (hb, 0, 0)),
        out_shape=jax.ShapeDtypeStruct((_H, _W, _C), jnp.float32),
        scratch_shapes=[
            pltpu.VMEM((_HB, _W, _C), jnp.float32),
            pltpu.VMEM((_HB, _W, _C), jnp.float32),
        ],
    )(g, row_emb, col_emb, token_type_emb, w2, b2)
    return out.reshape(_B, _T, _C)


# speedup vs baseline: 1.1008x; 1.1008x over previous
import jax
import jax.numpy as jnp
from jax.experimental import pallas as pl
from jax.experimental.pallas import tpu as pltpu

_B, _F, _H, _W, _C = 1, 64, 32, 32, 768
_T = _H * _W
_EPS = 1e-12

_FB = 8
_HB = 16
_NH = _H // _HB
_NFS = _F // _FB


def _body(g_ref, row_ref, col_ref, tte_ref, w_ref, b_ref, out_ref, acc_ref):
    fs = pl.program_id(1)
    hb = pl.program_id(0)
    part = g_ref[0]
    for i in range(1, _FB):
        part = part + g_ref[i]

    @pl.when(fs == 0)
    def _init():
        acc_ref[...] = part

    @pl.when(fs > 0)
    def _accum():
        acc_ref[...] += part

    @pl.when(fs == _NFS - 1)
    def _finish():
        x = acc_ref[...] * (1.0 / _F)
        row = row_ref[pl.ds(hb * _HB, _HB)]
        x = x + row[:, None, :] + col_ref[...][None, :, :]
        x = x + tte_ref[...][None, :, :]
        mu = jnp.mean(x, axis=-1, keepdims=True)
        var = jnp.mean(jnp.square(x - mu), axis=-1, keepdims=True)
        y = (x - mu) * jax.lax.rsqrt(var + _EPS)
        out_ref[...] = y * w_ref[...][None, :, :] + b_ref[...][None, :, :]


def kernel(grid, row_emb, col_emb, token_type_emb, ln_weight, ln_bias):
    g = grid.reshape(_F, _H, _W, _C)
    w2 = ln_weight.reshape(1, _C)
    b2 = ln_bias.reshape(1, _C)
    out = pl.pallas_call(
        _body,
        grid=(_NH, _NFS),
        in_specs=[
            pl.BlockSpec((_FB, _HB, _W, _C), lambda hb, fs: (fs, hb, 0, 0)),
            pl.BlockSpec((_H, _C), lambda hb, fs: (0, 0)),
            pl.BlockSpec((_W, _C), lambda hb, fs: (0, 0)),
            pl.BlockSpec((1, _C), lambda hb, fs: (0, 0)),
            pl.BlockSpec((1, _C), lambda hb, fs: (0, 0)),
            pl.BlockSpec((1, _C), lambda hb, fs: (0, 0)),
        ],
        out_specs=pl.BlockSpec((_HB, _W, _C), lambda hb, fs: (hb, 0, 0)),
        out_shape=jax.ShapeDtypeStruct((_H, _W, _C), jnp.float32),
        scratch_shapes=[pltpu.VMEM((_HB, _W, _C), jnp.float32)],
    )(g, row_emb, col_emb, token_type_emb, w2, b2)
    return out.reshape(_B, _T, _C)


# speedup vs baseline: 1.1158x; 1.0137x over previous
import jax
import jax.numpy as jnp
from jax.experimental import pallas as pl
from jax.experimental.pallas import tpu as pltpu

_B, _F, _H, _W, _C = 1, 64, 32, 32, 768
_T = _H * _W
_EPS = 1e-12

_CF = 4
_HB = 16
_NH = _H // _HB
_NCH = _F // _CF
_NBUF = 4


def _body(g_ref, row_ref, col_ref, tte_ref, w_ref, b_ref, out_ref,
          bufs_ref, sems):
    def fire(hb, c, slot):
        pltpu.make_async_copy(
            g_ref.at[pl.ds(c * _CF, _CF), pl.ds(hb * _HB, _HB)],
            bufs_ref.at[slot], sems.at[slot]).start()

    def wait(slot):
        pltpu.make_async_copy(
            g_ref.at[pl.ds(0, _CF), pl.ds(0, _HB)],
            bufs_ref.at[slot], sems.at[slot]).wait()

    for s in range(_NBUF):
        fire(0, s, s)

    for hb in range(_NH):
        for c in range(_NCH):
            slot = c % _NBUF
            wait(slot)
            part = bufs_ref[slot, 0]
            for i in range(1, _CF):
                part = part + bufs_ref[slot, i]
            dst = out_ref.at[pl.ds(hb * _HB, _HB)]
            if c == 0:
                dst[...] = part
            else:
                dst[...] += part
            nxt = hb * _NCH + c + _NBUF
            if nxt < _NH * _NCH:
                fire(nxt // _NCH, nxt % _NCH, slot)

        x = out_ref[pl.ds(hb * _HB, _HB)] * (1.0 / _F)
        row = row_ref[pl.ds(hb * _HB, _HB)]
        x = x + row[:, None, :] + col_ref[...][None, :, :]
        x = x + tte_ref[...][None, :, :]
        mu = jnp.mean(x, axis=-1, keepdims=True)
        var = jnp.mean(jnp.square(x - mu), axis=-1, keepdims=True)
        y = (x - mu) * jax.lax.rsqrt(var + _EPS)
        out_ref[pl.ds(hb * _HB, _HB)] = (
            y * w_ref[...][None, :, :] + b_ref[...][None, :, :])


def kernel(grid, row_emb, col_emb, token_type_emb, ln_weight, ln_bias):
    g = grid.reshape(_F, _H, _W, _C)
    w2 = ln_weight.reshape(1, _C)
    b2 = ln_bias.reshape(1, _C)
    out = pl.pallas_call(
        _body,
        in_specs=[
            pl.BlockSpec(memory_space=pl.ANY),
            pl.BlockSpec((_H, _C), lambda: (0, 0)),
            pl.BlockSpec((_W, _C), lambda: (0, 0)),
            pl.BlockSpec((1, _C), lambda: (0, 0)),
            pl.BlockSpec((1, _C), lambda: (0, 0)),
            pl.BlockSpec((1, _C), lambda: (0, 0)),
        ],
        out_specs=pl.BlockSpec((_H, _W, _C), lambda: (0, 0, 0)),
        out_shape=jax.ShapeDtypeStruct((_H, _W, _C), jnp.float32),
        scratch_shapes=[
            pltpu.VMEM((_NBUF, _CF, _HB, _W, _C), jnp.float32),
            pltpu.SemaphoreType.DMA((_NBUF,)),
        ],
    )(g, row_emb, col_emb, token_type_emb, w2, b2)
    return out.reshape(_B, _T, _C)
